# baseline (speedup 1.0000x reference)
import functools

import numpy as np
import jax
import jax.numpy as jnp
from jax.experimental import pallas as pl
from jax.experimental.pallas import tpu as pltpu

IN_C = 512
MID_C = 512
FEAT_STRIDE = 16
RATIOS = (0.5, 1.0, 2.0)
SCALES = (8, 16, 32)
IMG_H = 800
IMG_W = 800
HH = 50
WW = 50
N_PRE = 2000
N_POST = 300
NMS_T = 0.7
MIN_SIZE = 16.0
N_ANCHOR = 9
HW = HH * WW
HW_PAD = 2560
OUT_COLS = 128


def _anchor_base_np(base_size=16.0, ratios=RATIOS, anchor_scales=SCALES):
    py = base_size / 2.0
    px = base_size / 2.0
    ab = np.zeros((len(ratios) * len(anchor_scales), 4), dtype=np.float32)
    for i in range(len(ratios)):
        for j in range(len(anchor_scales)):
            h = base_size * anchor_scales[j] * np.sqrt(ratios[i])
            w = base_size * anchor_scales[j] * np.sqrt(1.0 / ratios[i])
            idx = i * len(anchor_scales) + j
            ab[idx, 0] = py - h / 2.0
            ab[idx, 1] = px - w / 2.0
            ab[idx, 2] = py + h / 2.0
            ab[idx, 3] = px + w / 2.0
    return ab


def _anchors_np():
    ab = _anchor_base_np()
    shift_y = np.arange(0, HH * FEAT_STRIDE, FEAT_STRIDE)
    shift_x = np.arange(0, WW * FEAT_STRIDE, FEAT_STRIDE)
    shift_x, shift_y = np.meshgrid(shift_x, shift_y)
    shift = np.stack((shift_y.ravel(), shift_x.ravel(),
                      shift_y.ravel(), shift_x.ravel()), axis=1)
    A = ab.shape[0]
    K = shift.shape[0]
    anchor = ab.reshape((1, A, 4)) + shift.reshape((1, K, 4)).transpose((1, 0, 2))
    return anchor.reshape((K * A, 4)).astype(np.float32)


_ANCHORS = _anchors_np()


def _conv_head_body(tap_ref, w_ref, w2_ref, b1_ref, b2_ref, out_ref, hacc):
    t = pl.program_id(0)

    @pl.when(t == 0)
    def _init():
        hacc[...] = jnp.zeros((HW_PAD, MID_C), jnp.float32)

    hacc[...] += jnp.dot(tap_ref[0], w_ref[0],
                         preferred_element_type=jnp.float32)

    @pl.when(t == 8)
    def _finish():
        h = jnp.maximum(hacc[...] + b1_ref[...], 0.0)
        out_ref[...] = jnp.dot(h, w2_ref[...],
                               preferred_element_type=jnp.float32) + b2_ref[...]


def _conv_head(x, W1, b1, Ws, bs, Wl, bl):
    xhwc = x[0].transpose(1, 2, 0)
    xp = jnp.pad(xhwc, ((1, 1), (1, 1), (0, 0)))
    taps = []
    for dy in range(3):
        for dx in range(3):
            sl = xp[dy:dy + HH, dx:dx + WW, :].reshape(HW, IN_C)
            taps.append(sl)
    taps = jnp.stack(taps, axis=0)
    taps = jnp.pad(taps, ((0, 0), (0, HW_PAD - HW), (0, 0)))

    wst = jnp.stack([W1[:, :, dy, dx].T for dy in range(3) for dx in range(3)],
                    axis=0)

    w2 = jnp.concatenate([Wl[:, :, 0, 0].T, Ws[:, :, 0, 0].T], axis=1)
    w2 = jnp.pad(w2, ((0, 0), (0, OUT_COLS - w2.shape[1])))
    b2 = jnp.concatenate([bl, bs])
    b2 = jnp.pad(b2, (0, OUT_COLS - b2.shape[0])).reshape(1, OUT_COLS)
    b1r = b1.reshape(1, MID_C)

    y2 = pl.pallas_call(
        _conv_head_body,
        grid=(9,),
        in_specs=[
            pl.BlockSpec((1, HW_PAD, IN_C), lambda t: (t, 0, 0)),
            pl.BlockSpec((1, IN_C, MID_C), lambda t: (t, 0, 0)),
            pl.BlockSpec((MID_C, OUT_COLS), lambda t: (0, 0)),
            pl.BlockSpec((1, MID_C), lambda t: (0, 0)),
            pl.BlockSpec((1, OUT_COLS), lambda t: (0, 0)),
        ],
        out_specs=pl.BlockSpec((HW_PAD, OUT_COLS), lambda t: (0, 0)),
        out_shape=jax.ShapeDtypeStruct((HW_PAD, OUT_COLS), jnp.float32),
        scratch_shapes=[pltpu.VMEM((HW_PAD, MID_C), jnp.float32)],
        compiler_params=pltpu.CompilerParams(
            dimension_semantics=("arbitrary",)),
    )(taps, wst, w2, b1r, b2)
    return y2


N_FLAT = 22528
N_ROWS = 176
N_BLK = 22
N_SORT = 2048
INT_MIN32 = -2147483648
UK_NEGINF = -2139095041
KEY_MASK = 0x7FFFFFFF
BIG = 1 << 30


def _fkey(x):
    b = jax.lax.bitcast_convert_type(x, jnp.int32)
    return jnp.where(b < 0, b ^ KEY_MASK, b)


def _pick_lane(row, l0):
    li = jax.lax.broadcasted_iota(jnp.int32, (1, 128), 1)
    return jnp.sum(jnp.where(li == l0, row, 0.0))


def _proposal_body(dy_ref, dx_ref, dh_ref, dw_ref, bg_ref, fg_ref,
                   ay1_ref, ax1_ref, ay2_ref, ax2_ref, imgsz_ref,
                   out_ref,
                   ry1_ref, rx1_ref, ry2_ref, rx2_ref, su_ref,
                   rby1_ref, rbx1_ref, rby2_ref, rbx2_ref, karr_ref,
                   ky1_ref, kx1_ref, ky2_ref, kx2_ref, kar_ref):
    img_h = imgsz_ref[0].astype(jnp.float32)
    img_w = imgsz_ref[1].astype(jnp.float32)

    a0, a1, a2, a3 = ay1_ref[...], ax1_ref[...], ay2_ref[...], ax2_ref[...]
    src_h = a2 - a0
    src_w = a3 - a1
    src_cy = a0 + 0.5 * src_h
    src_cx = a1 + 0.5 * src_w
    cy = dy_ref[...] * src_h + src_cy
    cx = dx_ref[...] * src_w + src_cx
    h = jnp.exp(dh_ref[...]) * src_h
    w = jnp.exp(dw_ref[...]) * src_w
    y1 = jnp.clip(cy - 0.5 * h, 0.0, img_h)
    x1 = jnp.clip(cx - 0.5 * w, 0.0, img_w)
    y2 = jnp.clip(cy + 0.5 * h, 0.0, img_h)
    x2 = jnp.clip(cx + 0.5 * w, 0.0, img_w)
    ry1_ref[...] = y1
    rx1_ref[...] = x1
    ry2_ref[...] = y2
    rx2_ref[...] = x2
    b, f = bg_ref[...], fg_ref[...]
    m = jnp.maximum(b, f)
    e0 = jnp.exp(b - m)
    e1 = jnp.exp(f - m)
    fgs = e1 / (e0 + e1)
    valid0 = jnp.logical_and(y2 - y1 >= MIN_SIZE, x2 - x1 >= MIN_SIZE)
    sc = jnp.where(valid0, fgs, -jnp.inf)
    row_i = jax.lax.broadcasted_iota(jnp.int32, (N_ROWS, 128), 0)
    lane_i = jax.lax.broadcasted_iota(jnp.int32, (N_ROWS, 128), 1)
    flat = row_i * 128 + lane_i
    real = flat < HW * N_ANCHOR
    su = jnp.where(real, _fkey(sc), INT_MIN32)
    su_ref[...] = su

    flat16 = (jax.lax.broadcasted_iota(jnp.int32, (16, 128), 0) * 128
              + jax.lax.broadcasted_iota(jnp.int32, (16, 128), 1))
    karr_ref[...] = jnp.full((16, 128), INT_MIN32, jnp.int32)
    zeros16 = jnp.zeros((16, 128), jnp.float32)
    ky1_ref[...] = zeros16
    kx1_ref[...] = zeros16
    ky2_ref[...] = zeros16
    kx2_ref[...] = zeros16
    kar_ref[...] = zeros16
    rby1_ref[...] = zeros16
    rbx1_ref[...] = zeros16
    rby2_ref[...] = zeros16
    rbx2_ref[...] = zeros16

    li8 = jax.lax.broadcasted_iota(jnp.int32, (8, 128), 1)
    si8 = jax.lax.broadcasted_iota(jnp.int32, (8, 128), 0)
    li1 = jax.lax.broadcasted_iota(jnp.int32, (1, 128), 1)
    r32 = jax.lax.broadcasted_iota(jnp.int32, (32, 128), 0)

    def pair_tree(v, f, axis, size):
        s = size // 2
        while s >= 1:
            v2 = jnp.roll(v, -s, axis=axis)
            f2 = jnp.roll(f, -s, axis=axis)
            take = jnp.logical_or(v2 > v, jnp.logical_and(v2 == v, f2 < f))
            v = jnp.where(take, v2, v)
            f = jnp.where(take, f2, f)
            s //= 2
        return v, f

    su3 = su.reshape(N_BLK, 8, 128)
    fl3 = flat.reshape(N_BLK, 8, 128)
    cv, cfv = pair_tree(su3, fl3, 1, 8)
    bm22 = cv[:, 0, :]
    cf22 = cfv[:, 0, :]
    bm0 = jnp.concatenate(
        [bm22, jnp.full((32 - N_BLK, 128), INT_MIN32, jnp.int32)], axis=0)
    cf0 = jnp.concatenate(
        [cf22, jnp.full((32 - N_BLK, 128), BIG, jnp.int32)], axis=0)

    def bcast_max(x, rows):
        s = rows // 2
        while s >= 1:
            x = jnp.maximum(x, jnp.roll(x, -s, axis=0))
            s //= 2
        x = x[0:1, :]
        s = 64
        while s >= 1:
            x = jnp.maximum(x, jnp.roll(x, -s, axis=1))
            s //= 2
        return x

    def lane_bcast(row, lmask):
        x = jnp.where(lmask, row, -jnp.inf)
        s = 64
        while s >= 1:
            x = jnp.maximum(x, jnp.roll(x, -s, axis=1))
            s //= 2
        return x

    def body(r, carry):
        nk_b, bm, cf = carry
        v, fsel = pair_tree(bm, cf, 0, 32)
        v, fsel = pair_tree(v[0:1, :], fsel[0:1, :], 1, 128)
        g_b = v
        f_b = fsel
        f0 = f_b[0, 0]
        rr = f0 // 128
        b8 = (rr // 8) * 8
        l0_b = f_b & 127
        lmask = li1 == l0_b
        su_ref[pl.ds(rr, 1), :] = jnp.where(lmask, INT_MIN32,
                                            su_ref[pl.ds(rr, 1), :])
        blk = su_ref[pl.ds(b8, 8), :]
        bflat = (b8 + si8) * 128 + li8
        nv, nf = pair_tree(blk, bflat, 0, 8)
        bmask = r32 == (f_b // 1024)
        bm = jnp.where(bmask, jnp.broadcast_to(nv[0:1, :], (32, 128)), bm)
        cf = jnp.where(bmask, jnp.broadcast_to(nf[0:1, :], (32, 128)), cf)
        by1 = lane_bcast(ry1_ref[pl.ds(rr, 1), :], lmask)
        bx1 = lane_bcast(rx1_ref[pl.ds(rr, 1), :], lmask)
        by2 = lane_bcast(ry2_ref[pl.ds(rr, 1), :], lmask)
        bx2 = lane_bcast(rx2_ref[pl.ds(rr, 1), :], lmask)
        barea = (by2 - by1) * (bx2 - bx1)
        tly = jnp.maximum(ky1_ref[...], by1)
        tlx = jnp.maximum(kx1_ref[...], bx1)
        bry = jnp.minimum(ky2_ref[...], by2)
        brx = jnp.minimum(kx2_ref[...], bx2)
        why = jnp.maximum(bry - tly, 0.0)
        whx = jnp.maximum(brx - tlx, 0.0)
        inter = why * whx
        iou = inter / (kar_ref[...] + barea - inter + 1e-9)
        v16 = jnp.where(iou > NMS_T, 1, 0)
        keep_b = bcast_max(v16, 16) == 0
        rmask = flat16 == r
        karr_ref[...] = jnp.where(rmask, jnp.where(keep_b, g_b, UK_NEGINF),
                                  karr_ref[...])
        rby1_ref[...] = jnp.where(rmask, by1, rby1_ref[...])
        rbx1_ref[...] = jnp.where(rmask, bx1, rbx1_ref[...])
        rby2_ref[...] = jnp.where(rmask, by2, rby2_ref[...])
        rbx2_ref[...] = jnp.where(rmask, bx2, rbx2_ref[...])
        amask = jnp.logical_and(flat16 == nk_b, keep_b)
        ky1_ref[...] = jnp.where(amask, by1, ky1_ref[...])
        kx1_ref[...] = jnp.where(amask, bx1, kx1_ref[...])
        ky2_ref[...] = jnp.where(amask, by2, ky2_ref[...])
        kx2_ref[...] = jnp.where(amask, bx2, kx2_ref[...])
        kar_ref[...] = jnp.where(amask, barea, kar_ref[...])
        nk_b = nk_b + jnp.where(keep_b, 1, 0)
        return nk_b, bm, cf

    jax.lax.fori_loop(0, N_PRE, body,
                      (jnp.zeros((1, 128), jnp.int32), bm0, cf0))

    def body2(t, dummy):
        k = karr_ref[...]
        g2 = jnp.max(k)
        fr = jnp.min(jnp.where(k == g2, flat16, BIG))
        rr2 = fr // 128
        ll2 = fr - rr2 * 128
        by1 = _pick_lane(rby1_ref[pl.ds(rr2, 1), :], ll2)
        bx1 = _pick_lane(rbx1_ref[pl.ds(rr2, 1), :], ll2)
        by2 = _pick_lane(rby2_ref[pl.ds(rr2, 1), :], ll2)
        bx2 = _pick_lane(rbx2_ref[pl.ds(rr2, 1), :], ll2)
        orow = jnp.where(li1 == 0, by1,
                         jnp.where(li1 == 1, bx1,
                                   jnp.where(li1 == 2, by2,
                                             jnp.where(li1 == 3, bx2, 0.0))))
        out_ref[pl.ds(t, 1), :] = orow
        karr_ref[...] = jnp.where(flat16 == fr, INT_MIN32, karr_ref[...])
        return dummy

    jax.lax.fori_loop(0, N_POST, body2, jnp.int32(0))


def _pad_rows(v):
    return jnp.pad(v.reshape(-1), (0, N_FLAT - HW * N_ANCHOR)).reshape(N_ROWS, 128)


_ANCH_PLANES = tuple(
    np.pad(_ANCHORS[:, c], (0, N_FLAT - HW * N_ANCHOR)).reshape(N_ROWS, 128)
    for c in range(4)
)


def _proposal_call(y2, img_size):
    locs2d = y2[:HW, 0:36]
    dy = _pad_rows(locs2d[:, 0::4])
    dx = _pad_rows(locs2d[:, 1::4])
    dh = _pad_rows(locs2d[:, 2::4])
    dw = _pad_rows(locs2d[:, 3::4])
    sc2d = y2[:HW, 36:54]
    bg = _pad_rows(sc2d[:, 0::2])
    fg = _pad_rows(sc2d[:, 1::2])
    anch = [jnp.asarray(p) for p in _ANCH_PLANES]

    big = pltpu.VMEM((N_ROWS, 128), jnp.float32)
    small = pltpu.VMEM((16, 128), jnp.float32)
    smalli = pltpu.VMEM((16, 128), jnp.int32)
    out = pl.pallas_call(
        _proposal_body,
        in_specs=[pl.BlockSpec((N_ROWS, 128), lambda: (0, 0))] * 10
        + [pl.BlockSpec(memory_space=pltpu.SMEM)],
        out_specs=pl.BlockSpec((304, 128), lambda: (0, 0)),
        out_shape=jax.ShapeDtypeStruct((304, 128), jnp.float32),
        scratch_shapes=[big, big, big, big,
                        pltpu.VMEM((N_ROWS, 128), jnp.int32),
                        small, small, small, small, smalli,
                        small, small, small, small, small],
    )(dy, dx, dh, dw, bg, fg, *anch, img_size)
    return out[:N_POST, 0:4]


def _pad_rows(v):
    return jnp.pad(v.reshape(-1), (0, N_FLAT - HW * N_ANCHOR)).reshape(N_ROWS, 128)


_ANCH_PLANES = tuple(
    np.pad(_ANCHORS[:, c], (0, N_FLAT - HW * N_ANCHOR)).reshape(N_ROWS, 128)
    for c in range(4)
)


def kernel(x, W1, b1, Ws, bs, Wl, bl, img_size):
    n = x.shape[0]
    y2 = _conv_head(x, W1, b1, Ws, bs, Wl, bl)
    locs = y2[:HW, 0:36]
    scores_raw = y2[:HW, 36:54]

    rpn_locs = locs.reshape(n, HH * WW * N_ANCHOR, 4)
    rpn_scores = scores_raw.reshape(n, HH * WW * N_ANCHOR, 2)

    rois = _proposal_call(y2, img_size)
    anchor = jnp.asarray(_ANCHORS)
    roi_indices = jnp.zeros((N_POST,), dtype=jnp.int32)
    return rpn_locs, rpn_scores, rois, roi_indices, anchor


# speedup vs baseline: 2.0229x; 2.0229x over previous
import functools

import numpy as np
import jax
import jax.numpy as jnp
from jax.experimental import pallas as pl
from jax.experimental.pallas import tpu as pltpu

IN_C = 512
MID_C = 512
FEAT_STRIDE = 16
RATIOS = (0.5, 1.0, 2.0)
SCALES = (8, 16, 32)
IMG_H = 800
IMG_W = 800
HH = 50
WW = 50
N_PRE = 2000
N_POST = 300
NMS_T = 0.7
MIN_SIZE = 16.0
N_ANCHOR = 9
HW = HH * WW
HW_PAD = 2560
OUT_COLS = 128


def _anchor_base_np(base_size=16.0, ratios=RATIOS, anchor_scales=SCALES):
    py = base_size / 2.0
    px = base_size / 2.0
    ab = np.zeros((len(ratios) * len(anchor_scales), 4), dtype=np.float32)
    for i in range(len(ratios)):
        for j in range(len(anchor_scales)):
            h = base_size * anchor_scales[j] * np.sqrt(ratios[i])
            w = base_size * anchor_scales[j] * np.sqrt(1.0 / ratios[i])
            idx = i * len(anchor_scales) + j
            ab[idx, 0] = py - h / 2.0
            ab[idx, 1] = px - w / 2.0
            ab[idx, 2] = py + h / 2.0
            ab[idx, 3] = px + w / 2.0
    return ab


def _anchors_np():
    ab = _anchor_base_np()
    shift_y = np.arange(0, HH * FEAT_STRIDE, FEAT_STRIDE)
    shift_x = np.arange(0, WW * FEAT_STRIDE, FEAT_STRIDE)
    shift_x, shift_y = np.meshgrid(shift_x, shift_y)
    shift = np.stack((shift_y.ravel(), shift_x.ravel(),
                      shift_y.ravel(), shift_x.ravel()), axis=1)
    A = ab.shape[0]
    K = shift.shape[0]
    anchor = ab.reshape((1, A, 4)) + shift.reshape((1, K, 4)).transpose((1, 0, 2))
    return anchor.reshape((K * A, 4)).astype(np.float32)


_ANCHORS = _anchors_np()


def _conv_head_body(tap_ref, w_ref, w2_ref, b1_ref, b2_ref, out_ref, hacc):
    t = pl.program_id(0)

    @pl.when(t == 0)
    def _init():
        hacc[...] = jnp.zeros((HW_PAD, MID_C), jnp.float32)

    hacc[...] += jnp.dot(tap_ref[0], w_ref[0],
                         preferred_element_type=jnp.float32)

    @pl.when(t == 8)
    def _finish():
        h = jnp.maximum(hacc[...] + b1_ref[...], 0.0)
        out_ref[...] = jnp.dot(h, w2_ref[...],
                               preferred_element_type=jnp.float32) + b2_ref[...]


def _conv_head(x, W1, b1, Ws, bs, Wl, bl):
    xhwc = x[0].transpose(1, 2, 0)
    xp = jnp.pad(xhwc, ((1, 1), (1, 1), (0, 0)))
    taps = []
    for dy in range(3):
        for dx in range(3):
            sl = xp[dy:dy + HH, dx:dx + WW, :].reshape(HW, IN_C)
            taps.append(sl)
    taps = jnp.stack(taps, axis=0)
    taps = jnp.pad(taps, ((0, 0), (0, HW_PAD - HW), (0, 0)))

    wst = jnp.stack([W1[:, :, dy, dx].T for dy in range(3) for dx in range(3)],
                    axis=0)

    w2 = jnp.concatenate([Wl[:, :, 0, 0].T, Ws[:, :, 0, 0].T], axis=1)
    w2 = jnp.pad(w2, ((0, 0), (0, OUT_COLS - w2.shape[1])))
    b2 = jnp.concatenate([bl, bs])
    b2 = jnp.pad(b2, (0, OUT_COLS - b2.shape[0])).reshape(1, OUT_COLS)
    b1r = b1.reshape(1, MID_C)

    y2 = pl.pallas_call(
        _conv_head_body,
        grid=(9,),
        in_specs=[
            pl.BlockSpec((1, HW_PAD, IN_C), lambda t: (t, 0, 0)),
            pl.BlockSpec((1, IN_C, MID_C), lambda t: (t, 0, 0)),
            pl.BlockSpec((MID_C, OUT_COLS), lambda t: (0, 0)),
            pl.BlockSpec((1, MID_C), lambda t: (0, 0)),
            pl.BlockSpec((1, OUT_COLS), lambda t: (0, 0)),
        ],
        out_specs=pl.BlockSpec((HW_PAD, OUT_COLS), lambda t: (0, 0)),
        out_shape=jax.ShapeDtypeStruct((HW_PAD, OUT_COLS), jnp.float32),
        scratch_shapes=[pltpu.VMEM((HW_PAD, MID_C), jnp.float32)],
        compiler_params=pltpu.CompilerParams(
            dimension_semantics=("arbitrary",)),
    )(taps, wst, w2, b1r, b2)
    return y2


N_FLAT = 22528
N_ROWS = 176
N_BLK = 22
N_SORT = 2048
INT_MIN32 = -2147483648
UK_NEGINF = -2139095041
KEY_MASK = 0x7FFFFFFF
BIG = 1 << 30


def _fkey(x):
    b = jax.lax.bitcast_convert_type(x, jnp.int32)
    return jnp.where(b < 0, b ^ KEY_MASK, b)


def _pick_lane(row, l0):
    li = jax.lax.broadcasted_iota(jnp.int32, (1, 128), 1)
    return jnp.sum(jnp.where(li == l0, row, 0.0))


def _proposal_body(dy_ref, dx_ref, dh_ref, dw_ref, bg_ref, fg_ref,
                   ay1_ref, ax1_ref, ay2_ref, ax2_ref, imgsz_ref,
                   out_ref,
                   ry1_ref, rx1_ref, ry2_ref, rx2_ref, su_ref,
                   rby1_ref, rbx1_ref, rby2_ref, rbx2_ref, karr_ref,
                   ky1_ref, kx1_ref, ky2_ref, kx2_ref, kar_ref):
    img_h = imgsz_ref[0].astype(jnp.float32)
    img_w = imgsz_ref[1].astype(jnp.float32)

    a0, a1, a2, a3 = ay1_ref[...], ax1_ref[...], ay2_ref[...], ax2_ref[...]
    src_h = a2 - a0
    src_w = a3 - a1
    src_cy = a0 + 0.5 * src_h
    src_cx = a1 + 0.5 * src_w
    cy = dy_ref[...] * src_h + src_cy
    cx = dx_ref[...] * src_w + src_cx
    h = jnp.exp(dh_ref[...]) * src_h
    w = jnp.exp(dw_ref[...]) * src_w
    y1 = jnp.clip(cy - 0.5 * h, 0.0, img_h)
    x1 = jnp.clip(cx - 0.5 * w, 0.0, img_w)
    y2 = jnp.clip(cy + 0.5 * h, 0.0, img_h)
    x2 = jnp.clip(cx + 0.5 * w, 0.0, img_w)
    ry1_ref[...] = y1
    rx1_ref[...] = x1
    ry2_ref[...] = y2
    rx2_ref[...] = x2
    b, f = bg_ref[...], fg_ref[...]
    m = jnp.maximum(b, f)
    e0 = jnp.exp(b - m)
    e1 = jnp.exp(f - m)
    fgs = e1 / (e0 + e1)
    valid0 = jnp.logical_and(y2 - y1 >= MIN_SIZE, x2 - x1 >= MIN_SIZE)
    sc = jnp.where(valid0, fgs, -jnp.inf)
    row_i = jax.lax.broadcasted_iota(jnp.int32, (N_ROWS, 128), 0)
    lane_i = jax.lax.broadcasted_iota(jnp.int32, (N_ROWS, 128), 1)
    flat = row_i * 128 + lane_i
    real = flat < HW * N_ANCHOR
    su = jnp.where(real, _fkey(sc), INT_MIN32)
    su_ref[...] = su

    flat16 = (jax.lax.broadcasted_iota(jnp.int32, (16, 128), 0) * 128
              + jax.lax.broadcasted_iota(jnp.int32, (16, 128), 1))
    karr_ref[...] = jnp.full((16, 128), INT_MIN32, jnp.int32)
    zeros16 = jnp.zeros((16, 128), jnp.float32)
    ky1_ref[...] = zeros16
    kx1_ref[...] = zeros16
    ky2_ref[...] = zeros16
    kx2_ref[...] = zeros16
    kar_ref[...] = zeros16
    rby1_ref[...] = zeros16
    rbx1_ref[...] = zeros16
    rby2_ref[...] = zeros16
    rbx2_ref[...] = zeros16

    li8 = jax.lax.broadcasted_iota(jnp.int32, (8, 128), 1)
    si8 = jax.lax.broadcasted_iota(jnp.int32, (8, 128), 0)
    li1 = jax.lax.broadcasted_iota(jnp.int32, (1, 128), 1)
    r22 = jax.lax.broadcasted_iota(jnp.int32, (N_BLK, 128), 0)

    su3 = su.reshape(N_BLK, 8, 128)
    fl3 = flat.reshape(N_BLK, 8, 128)
    bm0 = jnp.max(su3, axis=1)
    cf0 = jnp.min(jnp.where(su3 == bm0[:, None, :], fl3, BIG), axis=1)

    def body(r, carry):
        nk_b, bm, cf = carry
        g = jnp.max(bm)
        f0 = jnp.min(jnp.where(bm == g, cf, BIG))
        rr = f0 // 128
        l0 = f0 - rr * 128
        b0 = rr // 8
        b8 = b0 * 8
        lmask = li1 == l0
        su_ref[pl.ds(rr, 1), :] = jnp.where(lmask, INT_MIN32,
                                            su_ref[pl.ds(rr, 1), :])
        blk = su_ref[pl.ds(b8, 8), :]
        bflat = (b8 + si8) * 128 + li8
        nv = jnp.max(blk, axis=0, keepdims=True)
        nf = jnp.min(jnp.where(blk == nv, bflat, BIG), axis=0, keepdims=True)
        rowmask = r22 == b0
        bm = jnp.where(rowmask, jnp.broadcast_to(nv, (N_BLK, 128)), bm)
        cf = jnp.where(rowmask, jnp.broadcast_to(nf, (N_BLK, 128)), cf)
        by1 = _pick_lane(ry1_ref[pl.ds(rr, 1), :], l0)
        bx1 = _pick_lane(rx1_ref[pl.ds(rr, 1), :], l0)
        by2 = _pick_lane(ry2_ref[pl.ds(rr, 1), :], l0)
        bx2 = _pick_lane(rx2_ref[pl.ds(rr, 1), :], l0)
        barea = (by2 - by1) * (bx2 - bx1)
        tly = jnp.maximum(ky1_ref[...], by1)
        tlx = jnp.maximum(kx1_ref[...], bx1)
        bry = jnp.minimum(ky2_ref[...], by2)
        brx = jnp.minimum(kx2_ref[...], bx2)
        why = jnp.maximum(bry - tly, 0.0)
        whx = jnp.maximum(brx - tlx, 0.0)
        inter = why * whx
        iou = inter / (kar_ref[...] + barea - inter + 1e-9)
        viol = jnp.max(jnp.where(iou > NMS_T, 1.0, 0.0))
        keep = viol == 0.0
        rmask = flat16 == r
        karr_ref[...] = jnp.where(rmask, jnp.where(keep, g, UK_NEGINF),
                                  karr_ref[...])
        rby1_ref[...] = jnp.where(rmask, by1, rby1_ref[...])
        rbx1_ref[...] = jnp.where(rmask, bx1, rbx1_ref[...])
        rby2_ref[...] = jnp.where(rmask, by2, rby2_ref[...])
        rbx2_ref[...] = jnp.where(rmask, bx2, rbx2_ref[...])
        amask = jnp.logical_and(flat16 == nk_b, keep)
        ky1_ref[...] = jnp.where(amask, by1, ky1_ref[...])
        kx1_ref[...] = jnp.where(amask, bx1, kx1_ref[...])
        ky2_ref[...] = jnp.where(amask, by2, ky2_ref[...])
        kx2_ref[...] = jnp.where(amask, bx2, kx2_ref[...])
        kar_ref[...] = jnp.where(amask, barea, kar_ref[...])
        nk_b = nk_b + jnp.where(keep, 1, 0)
        return nk_b, bm, cf

    jax.lax.fori_loop(0, N_PRE, body,
                      (jnp.zeros((1, 128), jnp.int32), bm0, cf0))

    def body2(t, dummy):
        k = karr_ref[...]
        g2 = jnp.max(k)
        fr = jnp.min(jnp.where(k == g2, flat16, BIG))
        rr2 = fr // 128
        ll2 = fr - rr2 * 128
        by1 = _pick_lane(rby1_ref[pl.ds(rr2, 1), :], ll2)
        bx1 = _pick_lane(rbx1_ref[pl.ds(rr2, 1), :], ll2)
        by2 = _pick_lane(rby2_ref[pl.ds(rr2, 1), :], ll2)
        bx2 = _pick_lane(rbx2_ref[pl.ds(rr2, 1), :], ll2)
        orow = jnp.where(li1 == 0, by1,
                         jnp.where(li1 == 1, bx1,
                                   jnp.where(li1 == 2, by2,
                                             jnp.where(li1 == 3, bx2, 0.0))))
        out_ref[pl.ds(t, 1), :] = orow
        karr_ref[...] = jnp.where(flat16 == fr, INT_MIN32, karr_ref[...])
        return dummy

    jax.lax.fori_loop(0, N_POST, body2, jnp.int32(0))


def _pad_rows(v):
    return jnp.pad(v.reshape(-1), (0, N_FLAT - HW * N_ANCHOR)).reshape(N_ROWS, 128)


_ANCH_PLANES = tuple(
    np.pad(_ANCHORS[:, c], (0, N_FLAT - HW * N_ANCHOR)).reshape(N_ROWS, 128)
    for c in range(4)
)


def _proposal_call(y2, img_size):
    locs2d = y2[:HW, 0:36]
    dy = _pad_rows(locs2d[:, 0::4])
    dx = _pad_rows(locs2d[:, 1::4])
    dh = _pad_rows(locs2d[:, 2::4])
    dw = _pad_rows(locs2d[:, 3::4])
    sc2d = y2[:HW, 36:54]
    bg = _pad_rows(sc2d[:, 0::2])
    fg = _pad_rows(sc2d[:, 1::2])
    anch = [jnp.asarray(p) for p in _ANCH_PLANES]

    big = pltpu.VMEM((N_ROWS, 128), jnp.float32)
    small = pltpu.VMEM((16, 128), jnp.float32)
    smalli = pltpu.VMEM((16, 128), jnp.int32)
    out = pl.pallas_call(
        _proposal_body,
        in_specs=[pl.BlockSpec((N_ROWS, 128), lambda: (0, 0))] * 10
        + [pl.BlockSpec(memory_space=pltpu.SMEM)],
        out_specs=pl.BlockSpec((304, 128), lambda: (0, 0)),
        out_shape=jax.ShapeDtypeStruct((304, 128), jnp.float32),
        scratch_shapes=[big, big, big, big,
                        pltpu.VMEM((N_ROWS, 128), jnp.int32),
                        small, small, small, small, smalli,
                        small, small, small, small, small],
    )(dy, dx, dh, dw, bg, fg, *anch, img_size)
    return out[:N_POST, 0:4]


def _pad_rows(v):
    return jnp.pad(v.reshape(-1), (0, N_FLAT - HW * N_ANCHOR)).reshape(N_ROWS, 128)


_ANCH_PLANES = tuple(
    np.pad(_ANCHORS[:, c], (0, N_FLAT - HW * N_ANCHOR)).reshape(N_ROWS, 128)
    for c in range(4)
)


def kernel(x, W1, b1, Ws, bs, Wl, bl, img_size):
    n = x.shape[0]
    y2 = _conv_head(x, W1, b1, Ws, bs, Wl, bl)
    locs = y2[:HW, 0:36]
    scores_raw = y2[:HW, 36:54]

    rpn_locs = locs.reshape(n, HH * WW * N_ANCHOR, 4)
    rpn_scores = scores_raw.reshape(n, HH * WW * N_ANCHOR, 2)

    rois = _proposal_call(y2, img_size)
    anchor = jnp.asarray(_ANCHORS)
    roi_indices = jnp.zeros((N_POST,), dtype=jnp.int32)
    return rpn_locs, rpn_scores, rois, roi_indices, anchor


# speedup vs baseline: 2.2743x; 1.1243x over previous
import functools

import numpy as np
import jax
import jax.numpy as jnp
from jax.experimental import pallas as pl
from jax.experimental.pallas import tpu as pltpu

IN_C = 512
MID_C = 512
FEAT_STRIDE = 16
RATIOS = (0.5, 1.0, 2.0)
SCALES = (8, 16, 32)
IMG_H = 800
IMG_W = 800
HH = 50
WW = 50
N_PRE = 2000
N_POST = 300
NMS_T = 0.7
MIN_SIZE = 16.0
N_ANCHOR = 9
HW = HH * WW
HW_PAD = 2560
OUT_COLS = 128


def _anchor_base_np(base_size=16.0, ratios=RATIOS, anchor_scales=SCALES):
    py = base_size / 2.0
    px = base_size / 2.0
    ab = np.zeros((len(ratios) * len(anchor_scales), 4), dtype=np.float32)
    for i in range(len(ratios)):
        for j in range(len(anchor_scales)):
            h = base_size * anchor_scales[j] * np.sqrt(ratios[i])
            w = base_size * anchor_scales[j] * np.sqrt(1.0 / ratios[i])
            idx = i * len(anchor_scales) + j
            ab[idx, 0] = py - h / 2.0
            ab[idx, 1] = px - w / 2.0
            ab[idx, 2] = py + h / 2.0
            ab[idx, 3] = px + w / 2.0
    return ab


def _anchors_np():
    ab = _anchor_base_np()
    shift_y = np.arange(0, HH * FEAT_STRIDE, FEAT_STRIDE)
    shift_x = np.arange(0, WW * FEAT_STRIDE, FEAT_STRIDE)
    shift_x, shift_y = np.meshgrid(shift_x, shift_y)
    shift = np.stack((shift_y.ravel(), shift_x.ravel(),
                      shift_y.ravel(), shift_x.ravel()), axis=1)
    A = ab.shape[0]
    K = shift.shape[0]
    anchor = ab.reshape((1, A, 4)) + shift.reshape((1, K, 4)).transpose((1, 0, 2))
    return anchor.reshape((K * A, 4)).astype(np.float32)


_ANCHORS = _anchors_np()


def _conv_head_body(tap_ref, w_ref, w2_ref, b1_ref, b2_ref, out_ref, hacc):
    t = pl.program_id(0)

    @pl.when(t == 0)
    def _init():
        hacc[...] = jnp.zeros((HW_PAD, MID_C), jnp.float32)

    hacc[...] += jnp.dot(tap_ref[0], w_ref[0],
                         preferred_element_type=jnp.float32)

    @pl.when(t == 8)
    def _finish():
        h = jnp.maximum(hacc[...] + b1_ref[...], 0.0)
        out_ref[...] = jnp.dot(h, w2_ref[...],
                               preferred_element_type=jnp.float32) + b2_ref[...]


def _conv_head(x, W1, b1, Ws, bs, Wl, bl):
    xhwc = x[0].transpose(1, 2, 0)
    xp = jnp.pad(xhwc, ((1, 1), (1, 1), (0, 0)))
    taps = []
    for dy in range(3):
        for dx in range(3):
            sl = xp[dy:dy + HH, dx:dx + WW, :].reshape(HW, IN_C)
            taps.append(sl)
    taps = jnp.stack(taps, axis=0)
    taps = jnp.pad(taps, ((0, 0), (0, HW_PAD - HW), (0, 0)))

    wst = jnp.stack([W1[:, :, dy, dx].T for dy in range(3) for dx in range(3)],
                    axis=0)

    w2 = jnp.concatenate([Wl[:, :, 0, 0].T, Ws[:, :, 0, 0].T], axis=1)
    w2 = jnp.pad(w2, ((0, 0), (0, OUT_COLS - w2.shape[1])))
    b2 = jnp.concatenate([bl, bs])
    b2 = jnp.pad(b2, (0, OUT_COLS - b2.shape[0])).reshape(1, OUT_COLS)
    b1r = b1.reshape(1, MID_C)

    y2 = pl.pallas_call(
        _conv_head_body,
        grid=(9,),
        in_specs=[
            pl.BlockSpec((1, HW_PAD, IN_C), lambda t: (t, 0, 0)),
            pl.BlockSpec((1, IN_C, MID_C), lambda t: (t, 0, 0)),
            pl.BlockSpec((MID_C, OUT_COLS), lambda t: (0, 0)),
            pl.BlockSpec((1, MID_C), lambda t: (0, 0)),
            pl.BlockSpec((1, OUT_COLS), lambda t: (0, 0)),
        ],
        out_specs=pl.BlockSpec((HW_PAD, OUT_COLS), lambda t: (0, 0)),
        out_shape=jax.ShapeDtypeStruct((HW_PAD, OUT_COLS), jnp.float32),
        scratch_shapes=[pltpu.VMEM((HW_PAD, MID_C), jnp.float32)],
        compiler_params=pltpu.CompilerParams(
            dimension_semantics=("arbitrary",)),
    )(taps, wst, w2, b1r, b2)
    return y2


N_FLAT = 22528
N_ROWS = 176
N_BLK = 22
N_SORT = 2048
INT_MIN32 = -2147483648
UK_NEGINF = -2139095041
KEY_MASK = 0x7FFFFFFF
BIG = 1 << 30


def _fkey(x):
    b = jax.lax.bitcast_convert_type(x, jnp.int32)
    return jnp.where(b < 0, b ^ KEY_MASK, b)


def _pick_lane(row, l0):
    li = jax.lax.broadcasted_iota(jnp.int32, (1, 128), 1)
    return jnp.sum(jnp.where(li == l0, row, 0.0))


def _proposal_body(dy_ref, dx_ref, dh_ref, dw_ref, bg_ref, fg_ref,
                   ay1_ref, ax1_ref, ay2_ref, ax2_ref, imgsz_ref,
                   out_ref,
                   ry1_ref, rx1_ref, ry2_ref, rx2_ref, su_ref,
                   rby1_ref, rbx1_ref, rby2_ref, rbx2_ref, karr_ref,
                   ky1_ref, kx1_ref, ky2_ref, kx2_ref, kar_ref):
    img_h = imgsz_ref[0].astype(jnp.float32)
    img_w = imgsz_ref[1].astype(jnp.float32)

    a0, a1, a2, a3 = ay1_ref[...], ax1_ref[...], ay2_ref[...], ax2_ref[...]
    src_h = a2 - a0
    src_w = a3 - a1
    src_cy = a0 + 0.5 * src_h
    src_cx = a1 + 0.5 * src_w
    cy = dy_ref[...] * src_h + src_cy
    cx = dx_ref[...] * src_w + src_cx
    h = jnp.exp(dh_ref[...]) * src_h
    w = jnp.exp(dw_ref[...]) * src_w
    y1 = jnp.clip(cy - 0.5 * h, 0.0, img_h)
    x1 = jnp.clip(cx - 0.5 * w, 0.0, img_w)
    y2 = jnp.clip(cy + 0.5 * h, 0.0, img_h)
    x2 = jnp.clip(cx + 0.5 * w, 0.0, img_w)
    ry1_ref[...] = y1
    rx1_ref[...] = x1
    ry2_ref[...] = y2
    rx2_ref[...] = x2
    b, f = bg_ref[...], fg_ref[...]
    m = jnp.maximum(b, f)
    e0 = jnp.exp(b - m)
    e1 = jnp.exp(f - m)
    fgs = e1 / (e0 + e1)
    valid0 = jnp.logical_and(y2 - y1 >= MIN_SIZE, x2 - x1 >= MIN_SIZE)
    sc = jnp.where(valid0, fgs, -jnp.inf)
    row_i = jax.lax.broadcasted_iota(jnp.int32, (N_ROWS, 128), 0)
    lane_i = jax.lax.broadcasted_iota(jnp.int32, (N_ROWS, 128), 1)
    flat = row_i * 128 + lane_i
    real = flat < HW * N_ANCHOR
    su = jnp.where(real, _fkey(sc), INT_MIN32)
    su_ref[...] = su

    flat16 = (jax.lax.broadcasted_iota(jnp.int32, (16, 128), 0) * 128
              + jax.lax.broadcasted_iota(jnp.int32, (16, 128), 1))
    karr_ref[...] = jnp.full((16, 128), INT_MIN32, jnp.int32)
    zeros16 = jnp.zeros((16, 128), jnp.float32)
    ky1_ref[...] = zeros16
    kx1_ref[...] = zeros16
    ky2_ref[...] = zeros16
    kx2_ref[...] = zeros16
    kar_ref[...] = zeros16
    rby1_ref[...] = zeros16
    rbx1_ref[...] = zeros16
    rby2_ref[...] = zeros16
    rbx2_ref[...] = zeros16

    li8 = jax.lax.broadcasted_iota(jnp.int32, (8, 128), 1)
    si8 = jax.lax.broadcasted_iota(jnp.int32, (8, 128), 0)
    li1 = jax.lax.broadcasted_iota(jnp.int32, (1, 128), 1)
    r22 = jax.lax.broadcasted_iota(jnp.int32, (N_BLK, 128), 0)

    su3 = su.reshape(N_BLK, 8, 128)
    fl3 = flat.reshape(N_BLK, 8, 128)
    bm0 = jnp.max(su3, axis=1)
    cf0 = jnp.min(jnp.where(su3 == bm0[:, None, :], fl3, BIG), axis=1)

    def body(r, carry):
        nk_b, bm, cf = carry
        g = jnp.max(bm)
        f0 = jnp.min(jnp.where(bm == g, cf, BIG))
        rr = f0 // 128
        l0 = f0 - rr * 128
        b0 = rr // 8
        b8 = b0 * 8
        lmask = li1 == l0
        su_ref[pl.ds(rr, 1), :] = jnp.where(lmask, INT_MIN32,
                                            su_ref[pl.ds(rr, 1), :])
        blk = su_ref[pl.ds(b8, 8), :]
        bflat = (b8 + si8) * 128 + li8
        nv = jnp.max(blk, axis=0, keepdims=True)
        nf = jnp.min(jnp.where(blk == nv, bflat, BIG), axis=0, keepdims=True)
        rowmask = r22 == b0
        bm = jnp.where(rowmask, jnp.broadcast_to(nv, (N_BLK, 128)), bm)
        cf = jnp.where(rowmask, jnp.broadcast_to(nf, (N_BLK, 128)), cf)
        by1 = _pick_lane(ry1_ref[pl.ds(rr, 1), :], l0)
        bx1 = _pick_lane(rx1_ref[pl.ds(rr, 1), :], l0)
        by2 = _pick_lane(ry2_ref[pl.ds(rr, 1), :], l0)
        bx2 = _pick_lane(rx2_ref[pl.ds(rr, 1), :], l0)
        barea = (by2 - by1) * (bx2 - bx1)
        tly = jnp.maximum(ky1_ref[...], by1)
        tlx = jnp.maximum(kx1_ref[...], bx1)
        bry = jnp.minimum(ky2_ref[...], by2)
        brx = jnp.minimum(kx2_ref[...], bx2)
        why = jnp.maximum(bry - tly, 0.0)
        whx = jnp.maximum(brx - tlx, 0.0)
        inter = why * whx
        iou = inter / (kar_ref[...] + barea - inter + 1e-9)
        viol = jnp.max(jnp.where(iou > NMS_T, 1.0, 0.0))
        keep = viol == 0.0
        rmask = flat16 == r
        karr_ref[...] = jnp.where(rmask, jnp.where(keep, g, UK_NEGINF),
                                  karr_ref[...])
        rby1_ref[...] = jnp.where(rmask, by1, rby1_ref[...])
        rbx1_ref[...] = jnp.where(rmask, bx1, rbx1_ref[...])
        rby2_ref[...] = jnp.where(rmask, by2, rby2_ref[...])
        rbx2_ref[...] = jnp.where(rmask, bx2, rbx2_ref[...])
        amask = jnp.logical_and(flat16 == nk_b, keep)
        ky1_ref[...] = jnp.where(amask, by1, ky1_ref[...])
        kx1_ref[...] = jnp.where(amask, bx1, kx1_ref[...])
        ky2_ref[...] = jnp.where(amask, by2, ky2_ref[...])
        kx2_ref[...] = jnp.where(amask, bx2, kx2_ref[...])
        kar_ref[...] = jnp.where(amask, barea, kar_ref[...])
        nk_b = nk_b + jnp.where(keep, 1, 0)
        return nk_b, bm, cf

    def body2x(i, carry):
        carry = body(2 * i, carry)
        return body(2 * i + 1, carry)

    jax.lax.fori_loop(0, N_PRE // 2, body2x,
                      (jnp.zeros((1, 128), jnp.int32), bm0, cf0))

    def body2(t, dummy):
        k = karr_ref[...]
        g2 = jnp.max(k)
        fr = jnp.min(jnp.where(k == g2, flat16, BIG))
        rr2 = fr // 128
        ll2 = fr - rr2 * 128
        by1 = _pick_lane(rby1_ref[pl.ds(rr2, 1), :], ll2)
        bx1 = _pick_lane(rbx1_ref[pl.ds(rr2, 1), :], ll2)
        by2 = _pick_lane(rby2_ref[pl.ds(rr2, 1), :], ll2)
        bx2 = _pick_lane(rbx2_ref[pl.ds(rr2, 1), :], ll2)
        orow = jnp.where(li1 == 0, by1,
                         jnp.where(li1 == 1, bx1,
                                   jnp.where(li1 == 2, by2,
                                             jnp.where(li1 == 3, bx2, 0.0))))
        out_ref[pl.ds(t, 1), :] = orow
        karr_ref[...] = jnp.where(flat16 == fr, INT_MIN32, karr_ref[...])
        return dummy

    jax.lax.fori_loop(0, N_POST, body2, jnp.int32(0))


def _pad_rows(v):
    return jnp.pad(v.reshape(-1), (0, N_FLAT - HW * N_ANCHOR)).reshape(N_ROWS, 128)


_ANCH_PLANES = tuple(
    np.pad(_ANCHORS[:, c], (0, N_FLAT - HW * N_ANCHOR)).reshape(N_ROWS, 128)
    for c in range(4)
)


def _proposal_call(y2, img_size):
    locs2d = y2[:HW, 0:36]
    dy = _pad_rows(locs2d[:, 0::4])
    dx = _pad_rows(locs2d[:, 1::4])
    dh = _pad_rows(locs2d[:, 2::4])
    dw = _pad_rows(locs2d[:, 3::4])
    sc2d = y2[:HW, 36:54]
    bg = _pad_rows(sc2d[:, 0::2])
    fg = _pad_rows(sc2d[:, 1::2])
    anch = [jnp.asarray(p) for p in _ANCH_PLANES]

    big = pltpu.VMEM((N_ROWS, 128), jnp.float32)
    small = pltpu.VMEM((16, 128), jnp.float32)
    smalli = pltpu.VMEM((16, 128), jnp.int32)
    out = pl.pallas_call(
        _proposal_body,
        in_specs=[pl.BlockSpec((N_ROWS, 128), lambda: (0, 0))] * 10
        + [pl.BlockSpec(memory_space=pltpu.SMEM)],
        out_specs=pl.BlockSpec((304, 128), lambda: (0, 0)),
        out_shape=jax.ShapeDtypeStruct((304, 128), jnp.float32),
        scratch_shapes=[big, big, big, big,
                        pltpu.VMEM((N_ROWS, 128), jnp.int32),
                        small, small, small, small, smalli,
                        small, small, small, small, small],
    )(dy, dx, dh, dw, bg, fg, *anch, img_size)
    return out[:N_POST, 0:4]


def _pad_rows(v):
    return jnp.pad(v.reshape(-1), (0, N_FLAT - HW * N_ANCHOR)).reshape(N_ROWS, 128)


_ANCH_PLANES = tuple(
    np.pad(_ANCHORS[:, c], (0, N_FLAT - HW * N_ANCHOR)).reshape(N_ROWS, 128)
    for c in range(4)
)


def kernel(x, W1, b1, Ws, bs, Wl, bl, img_size):
    n = x.shape[0]
    y2 = _conv_head(x, W1, b1, Ws, bs, Wl, bl)
    locs = y2[:HW, 0:36]
    scores_raw = y2[:HW, 36:54]

    rpn_locs = locs.reshape(n, HH * WW * N_ANCHOR, 4)
    rpn_scores = scores_raw.reshape(n, HH * WW * N_ANCHOR, 2)

    rois = _proposal_call(y2, img_size)
    anchor = jnp.asarray(_ANCHORS)
    roi_indices = jnp.zeros((N_POST,), dtype=jnp.int32)
    return rpn_locs, rpn_scores, rois, roi_indices, anchor


# speedup vs baseline: 2.4839x; 1.0922x over previous
import functools

import numpy as np
import jax
import jax.numpy as jnp
from jax.experimental import pallas as pl
from jax.experimental.pallas import tpu as pltpu

IN_C = 512
MID_C = 512
FEAT_STRIDE = 16
RATIOS = (0.5, 1.0, 2.0)
SCALES = (8, 16, 32)
IMG_H = 800
IMG_W = 800
HH = 50
WW = 50
N_PRE = 2000
N_POST = 300
NMS_T = 0.7
MIN_SIZE = 16.0
N_ANCHOR = 9
HW = HH * WW
HW_PAD = 2560
OUT_COLS = 128


def _anchor_base_np(base_size=16.0, ratios=RATIOS, anchor_scales=SCALES):
    py = base_size / 2.0
    px = base_size / 2.0
    ab = np.zeros((len(ratios) * len(anchor_scales), 4), dtype=np.float32)
    for i in range(len(ratios)):
        for j in range(len(anchor_scales)):
            h = base_size * anchor_scales[j] * np.sqrt(ratios[i])
            w = base_size * anchor_scales[j] * np.sqrt(1.0 / ratios[i])
            idx = i * len(anchor_scales) + j
            ab[idx, 0] = py - h / 2.0
            ab[idx, 1] = px - w / 2.0
            ab[idx, 2] = py + h / 2.0
            ab[idx, 3] = px + w / 2.0
    return ab


def _anchors_np():
    ab = _anchor_base_np()
    shift_y = np.arange(0, HH * FEAT_STRIDE, FEAT_STRIDE)
    shift_x = np.arange(0, WW * FEAT_STRIDE, FEAT_STRIDE)
    shift_x, shift_y = np.meshgrid(shift_x, shift_y)
    shift = np.stack((shift_y.ravel(), shift_x.ravel(),
                      shift_y.ravel(), shift_x.ravel()), axis=1)
    A = ab.shape[0]
    K = shift.shape[0]
    anchor = ab.reshape((1, A, 4)) + shift.reshape((1, K, 4)).transpose((1, 0, 2))
    return anchor.reshape((K * A, 4)).astype(np.float32)


_ANCHORS = _anchors_np()


def _conv_head_body(tap_ref, w_ref, w2_ref, b1_ref, b2_ref, out_ref, hacc):
    t = pl.program_id(0)

    @pl.when(t == 0)
    def _init():
        hacc[...] = jnp.zeros((HW_PAD, MID_C), jnp.float32)

    hacc[...] += jnp.dot(tap_ref[0], w_ref[0],
                         preferred_element_type=jnp.float32)

    @pl.when(t == 8)
    def _finish():
        h = jnp.maximum(hacc[...] + b1_ref[...], 0.0)
        out_ref[...] = jnp.dot(h, w2_ref[...],
                               preferred_element_type=jnp.float32) + b2_ref[...]


def _conv_head(x, W1, b1, Ws, bs, Wl, bl):
    xhwc = x[0].transpose(1, 2, 0)
    xp = jnp.pad(xhwc, ((1, 1), (1, 1), (0, 0)))
    taps = []
    for dy in range(3):
        for dx in range(3):
            sl = xp[dy:dy + HH, dx:dx + WW, :].reshape(HW, IN_C)
            taps.append(sl)
    taps = jnp.stack(taps, axis=0)
    taps = jnp.pad(taps, ((0, 0), (0, HW_PAD - HW), (0, 0)))

    wst = jnp.stack([W1[:, :, dy, dx].T for dy in range(3) for dx in range(3)],
                    axis=0)

    w2 = jnp.concatenate([Wl[:, :, 0, 0].T, Ws[:, :, 0, 0].T], axis=1)
    w2 = jnp.pad(w2, ((0, 0), (0, OUT_COLS - w2.shape[1])))
    b2 = jnp.concatenate([bl, bs])
    b2 = jnp.pad(b2, (0, OUT_COLS - b2.shape[0])).reshape(1, OUT_COLS)
    b1r = b1.reshape(1, MID_C)

    y2 = pl.pallas_call(
        _conv_head_body,
        grid=(9,),
        in_specs=[
            pl.BlockSpec((1, HW_PAD, IN_C), lambda t: (t, 0, 0)),
            pl.BlockSpec((1, IN_C, MID_C), lambda t: (t, 0, 0)),
            pl.BlockSpec((MID_C, OUT_COLS), lambda t: (0, 0)),
            pl.BlockSpec((1, MID_C), lambda t: (0, 0)),
            pl.BlockSpec((1, OUT_COLS), lambda t: (0, 0)),
        ],
        out_specs=pl.BlockSpec((HW_PAD, OUT_COLS), lambda t: (0, 0)),
        out_shape=jax.ShapeDtypeStruct((HW_PAD, OUT_COLS), jnp.float32),
        scratch_shapes=[pltpu.VMEM((HW_PAD, MID_C), jnp.float32)],
        compiler_params=pltpu.CompilerParams(
            dimension_semantics=("arbitrary",)),
    )(taps, wst, w2, b1r, b2)
    return y2


N_FLAT = 22528
N_ROWS = 176
N_BLK = 22
N_SORT = 2048
INT_MIN32 = -2147483648
UK_NEGINF = -2139095041
KEY_MASK = 0x7FFFFFFF
BIG = 1 << 30


def _fkey(x):
    b = jax.lax.bitcast_convert_type(x, jnp.int32)
    return jnp.where(b < 0, b ^ KEY_MASK, b)


def _pick_lane(row, l0):
    li = jax.lax.broadcasted_iota(jnp.int32, (1, 128), 1)
    return jnp.sum(jnp.where(li == l0, row, 0.0))


def _proposal_body(dy_ref, dx_ref, dh_ref, dw_ref, bg_ref, fg_ref,
                   ay1_ref, ax1_ref, ay2_ref, ax2_ref, imgsz_ref,
                   out_ref,
                   ry1_ref, rx1_ref, ry2_ref, rx2_ref, su_ref,
                   rby1_ref, rbx1_ref, rby2_ref, rbx2_ref, karr_ref,
                   ky1_ref, kx1_ref, ky2_ref, kx2_ref, kar_ref):
    img_h = imgsz_ref[0].astype(jnp.float32)
    img_w = imgsz_ref[1].astype(jnp.float32)

    a0, a1, a2, a3 = ay1_ref[...], ax1_ref[...], ay2_ref[...], ax2_ref[...]
    src_h = a2 - a0
    src_w = a3 - a1
    src_cy = a0 + 0.5 * src_h
    src_cx = a1 + 0.5 * src_w
    cy = dy_ref[...] * src_h + src_cy
    cx = dx_ref[...] * src_w + src_cx
    h = jnp.exp(dh_ref[...]) * src_h
    w = jnp.exp(dw_ref[...]) * src_w
    y1 = jnp.clip(cy - 0.5 * h, 0.0, img_h)
    x1 = jnp.clip(cx - 0.5 * w, 0.0, img_w)
    y2 = jnp.clip(cy + 0.5 * h, 0.0, img_h)
    x2 = jnp.clip(cx + 0.5 * w, 0.0, img_w)
    ry1_ref[...] = y1
    rx1_ref[...] = x1
    ry2_ref[...] = y2
    rx2_ref[...] = x2
    b, f = bg_ref[...], fg_ref[...]
    m = jnp.maximum(b, f)
    e0 = jnp.exp(b - m)
    e1 = jnp.exp(f - m)
    fgs = e1 / (e0 + e1)
    valid0 = jnp.logical_and(y2 - y1 >= MIN_SIZE, x2 - x1 >= MIN_SIZE)
    sc = jnp.where(valid0, fgs, -jnp.inf)
    row_i = jax.lax.broadcasted_iota(jnp.int32, (N_ROWS, 128), 0)
    lane_i = jax.lax.broadcasted_iota(jnp.int32, (N_ROWS, 128), 1)
    flat = row_i * 128 + lane_i
    real = flat < HW * N_ANCHOR
    su = jnp.where(real, _fkey(sc), INT_MIN32)
    su_ref[...] = su

    flat16 = (jax.lax.broadcasted_iota(jnp.int32, (16, 128), 0) * 128
              + jax.lax.broadcasted_iota(jnp.int32, (16, 128), 1))
    karr_ref[...] = jnp.full((16, 128), INT_MIN32, jnp.int32)
    zeros16 = jnp.zeros((16, 128), jnp.float32)
    ky1_ref[...] = zeros16
    kx1_ref[...] = zeros16
    ky2_ref[...] = zeros16
    kx2_ref[...] = zeros16
    kar_ref[...] = zeros16
    rby1_ref[...] = zeros16
    rbx1_ref[...] = zeros16
    rby2_ref[...] = zeros16
    rbx2_ref[...] = zeros16

    li8 = jax.lax.broadcasted_iota(jnp.int32, (8, 128), 1)
    si8 = jax.lax.broadcasted_iota(jnp.int32, (8, 128), 0)
    li1 = jax.lax.broadcasted_iota(jnp.int32, (1, 128), 1)
    r22 = jax.lax.broadcasted_iota(jnp.int32, (N_BLK, 128), 0)

    su3 = su.reshape(N_BLK, 8, 128)
    fl3 = flat.reshape(N_BLK, 8, 128)
    bm0 = jnp.max(su3, axis=1)
    cf0 = jnp.min(jnp.where(su3 == bm0[:, None, :], fl3, BIG), axis=1)

    def body(r, carry):
        nk_b, bm, cf = carry
        g = jnp.max(bm)
        f0 = jnp.min(jnp.where(bm == g, cf, BIG))
        rr = f0 // 128
        l0 = f0 - rr * 128
        b0 = rr // 8
        b8 = b0 * 8
        lmask = li1 == l0
        su_ref[pl.ds(rr, 1), :] = jnp.where(lmask, INT_MIN32,
                                            su_ref[pl.ds(rr, 1), :])
        blk = su_ref[pl.ds(b8, 8), :]
        bflat = (b8 + si8) * 128 + li8
        nv = jnp.max(blk, axis=0, keepdims=True)
        nf = jnp.min(jnp.where(blk == nv, bflat, BIG), axis=0, keepdims=True)
        rowmask = r22 == b0
        bm = jnp.where(rowmask, jnp.broadcast_to(nv, (N_BLK, 128)), bm)
        cf = jnp.where(rowmask, jnp.broadcast_to(nf, (N_BLK, 128)), cf)
        by1 = _pick_lane(ry1_ref[pl.ds(rr, 1), :], l0)
        bx1 = _pick_lane(rx1_ref[pl.ds(rr, 1), :], l0)
        by2 = _pick_lane(ry2_ref[pl.ds(rr, 1), :], l0)
        bx2 = _pick_lane(rx2_ref[pl.ds(rr, 1), :], l0)
        barea = (by2 - by1) * (bx2 - bx1)
        tly = jnp.maximum(ky1_ref[...], by1)
        tlx = jnp.maximum(kx1_ref[...], bx1)
        bry = jnp.minimum(ky2_ref[...], by2)
        brx = jnp.minimum(kx2_ref[...], bx2)
        why = jnp.maximum(bry - tly, 0.0)
        whx = jnp.maximum(brx - tlx, 0.0)
        inter = why * whx
        iou = inter / (kar_ref[...] + barea - inter + 1e-9)
        viol = jnp.max(jnp.where(iou > NMS_T, 1.0, 0.0))
        keep = viol == 0.0
        rmask = flat16 == r
        karr_ref[...] = jnp.where(rmask, jnp.where(keep, g, UK_NEGINF),
                                  karr_ref[...])
        rby1_ref[...] = jnp.where(rmask, by1, rby1_ref[...])
        rbx1_ref[...] = jnp.where(rmask, bx1, rbx1_ref[...])
        rby2_ref[...] = jnp.where(rmask, by2, rby2_ref[...])
        rbx2_ref[...] = jnp.where(rmask, bx2, rbx2_ref[...])
        amask = jnp.logical_and(flat16 == nk_b, keep)
        ky1_ref[...] = jnp.where(amask, by1, ky1_ref[...])
        kx1_ref[...] = jnp.where(amask, bx1, kx1_ref[...])
        ky2_ref[...] = jnp.where(amask, by2, ky2_ref[...])
        kx2_ref[...] = jnp.where(amask, bx2, kx2_ref[...])
        kar_ref[...] = jnp.where(amask, barea, kar_ref[...])
        nk_b = nk_b + jnp.where(keep, 1, 0)
        return nk_b, bm, cf

    def body4x(i, carry):
        carry = body(4 * i, carry)
        carry = body(4 * i + 1, carry)
        carry = body(4 * i + 2, carry)
        return body(4 * i + 3, carry)

    jax.lax.fori_loop(0, N_PRE // 4, body4x,
                      (jnp.zeros((1, 128), jnp.int32), bm0, cf0))

    def body2(t, dummy):
        k = karr_ref[...]
        g2 = jnp.max(k)
        fr = jnp.min(jnp.where(k == g2, flat16, BIG))
        rr2 = fr // 128
        ll2 = fr - rr2 * 128
        by1 = _pick_lane(rby1_ref[pl.ds(rr2, 1), :], ll2)
        bx1 = _pick_lane(rbx1_ref[pl.ds(rr2, 1), :], ll2)
        by2 = _pick_lane(rby2_ref[pl.ds(rr2, 1), :], ll2)
        bx2 = _pick_lane(rbx2_ref[pl.ds(rr2, 1), :], ll2)
        orow = jnp.where(li1 == 0, by1,
                         jnp.where(li1 == 1, bx1,
                                   jnp.where(li1 == 2, by2,
                                             jnp.where(li1 == 3, bx2, 0.0))))
        out_ref[pl.ds(t, 1), :] = orow
        karr_ref[...] = jnp.where(flat16 == fr, INT_MIN32, karr_ref[...])
        return dummy

    def body2x2(i, dummy):
        body2(2 * i, dummy)
        return body2(2 * i + 1, dummy)

    jax.lax.fori_loop(0, N_POST // 2, body2x2, jnp.int32(0))


def _pad_rows(v):
    return jnp.pad(v.reshape(-1), (0, N_FLAT - HW * N_ANCHOR)).reshape(N_ROWS, 128)


_ANCH_PLANES = tuple(
    np.pad(_ANCHORS[:, c], (0, N_FLAT - HW * N_ANCHOR)).reshape(N_ROWS, 128)
    for c in range(4)
)


def _proposal_call(y2, img_size):
    locs2d = y2[:HW, 0:36]
    dy = _pad_rows(locs2d[:, 0::4])
    dx = _pad_rows(locs2d[:, 1::4])
    dh = _pad_rows(locs2d[:, 2::4])
    dw = _pad_rows(locs2d[:, 3::4])
    sc2d = y2[:HW, 36:54]
    bg = _pad_rows(sc2d[:, 0::2])
    fg = _pad_rows(sc2d[:, 1::2])
    anch = [jnp.asarray(p) for p in _ANCH_PLANES]

    big = pltpu.VMEM((N_ROWS, 128), jnp.float32)
    small = pltpu.VMEM((16, 128), jnp.float32)
    smalli = pltpu.VMEM((16, 128), jnp.int32)
    out = pl.pallas_call(
        _proposal_body,
        in_specs=[pl.BlockSpec((N_ROWS, 128), lambda: (0, 0))] * 10
        + [pl.BlockSpec(memory_space=pltpu.SMEM)],
        out_specs=pl.BlockSpec((304, 128), lambda: (0, 0)),
        out_shape=jax.ShapeDtypeStruct((304, 128), jnp.float32),
        scratch_shapes=[big, big, big, big,
                        pltpu.VMEM((N_ROWS, 128), jnp.int32),
                        small, small, small, small, smalli,
                        small, small, small, small, small],
    )(dy, dx, dh, dw, bg, fg, *anch, img_size)
    return out[:N_POST, 0:4]


def _pad_rows(v):
    return jnp.pad(v.reshape(-1), (0, N_FLAT - HW * N_ANCHOR)).reshape(N_ROWS, 128)


_ANCH_PLANES = tuple(
    np.pad(_ANCHORS[:, c], (0, N_FLAT - HW * N_ANCHOR)).reshape(N_ROWS, 128)
    for c in range(4)
)


def kernel(x, W1, b1, Ws, bs, Wl, bl, img_size):
    n = x.shape[0]
    y2 = _conv_head(x, W1, b1, Ws, bs, Wl, bl)
    locs = y2[:HW, 0:36]
    scores_raw = y2[:HW, 36:54]

    rpn_locs = locs.reshape(n, HH * WW * N_ANCHOR, 4)
    rpn_scores = scores_raw.reshape(n, HH * WW * N_ANCHOR, 2)

    rois = _proposal_call(y2, img_size)
    anchor = jnp.asarray(_ANCHORS)
    roi_indices = jnp.zeros((N_POST,), dtype=jnp.int32)
    return rpn_locs, rpn_scores, rois, roi_indices, anchor


# speedup vs baseline: 2.5939x; 1.0443x over previous
import functools

import numpy as np
import jax
import jax.numpy as jnp
from jax.experimental import pallas as pl
from jax.experimental.pallas import tpu as pltpu

IN_C = 512
MID_C = 512
FEAT_STRIDE = 16
RATIOS = (0.5, 1.0, 2.0)
SCALES = (8, 16, 32)
IMG_H = 800
IMG_W = 800
HH = 50
WW = 50
N_PRE = 2000
N_POST = 300
NMS_T = 0.7
MIN_SIZE = 16.0
N_ANCHOR = 9
HW = HH * WW
HW_PAD = 2560
OUT_COLS = 128


def _anchor_base_np(base_size=16.0, ratios=RATIOS, anchor_scales=SCALES):
    py = base_size / 2.0
    px = base_size / 2.0
    ab = np.zeros((len(ratios) * len(anchor_scales), 4), dtype=np.float32)
    for i in range(len(ratios)):
        for j in range(len(anchor_scales)):
            h = base_size * anchor_scales[j] * np.sqrt(ratios[i])
            w = base_size * anchor_scales[j] * np.sqrt(1.0 / ratios[i])
            idx = i * len(anchor_scales) + j
            ab[idx, 0] = py - h / 2.0
            ab[idx, 1] = px - w / 2.0
            ab[idx, 2] = py + h / 2.0
            ab[idx, 3] = px + w / 2.0
    return ab


def _anchors_np():
    ab = _anchor_base_np()
    shift_y = np.arange(0, HH * FEAT_STRIDE, FEAT_STRIDE)
    shift_x = np.arange(0, WW * FEAT_STRIDE, FEAT_STRIDE)
    shift_x, shift_y = np.meshgrid(shift_x, shift_y)
    shift = np.stack((shift_y.ravel(), shift_x.ravel(),
                      shift_y.ravel(), shift_x.ravel()), axis=1)
    A = ab.shape[0]
    K = shift.shape[0]
    anchor = ab.reshape((1, A, 4)) + shift.reshape((1, K, 4)).transpose((1, 0, 2))
    return anchor.reshape((K * A, 4)).astype(np.float32)


_ANCHORS = _anchors_np()


def _conv_head_body(tap_ref, w_ref, w2_ref, b1_ref, b2_ref, out_ref, hacc):
    t = pl.program_id(0)

    @pl.when(t == 0)
    def _init():
        hacc[...] = jnp.zeros((HW_PAD, MID_C), jnp.float32)

    hacc[...] += jnp.dot(tap_ref[0], w_ref[0],
                         preferred_element_type=jnp.float32)

    @pl.when(t == 8)
    def _finish():
        h = jnp.maximum(hacc[...] + b1_ref[...], 0.0)
        out_ref[...] = jnp.dot(h, w2_ref[...],
                               preferred_element_type=jnp.float32) + b2_ref[...]


def _conv_head(x, W1, b1, Ws, bs, Wl, bl):
    xhwc = x[0].transpose(1, 2, 0)
    xp = jnp.pad(xhwc, ((1, 1), (1, 1), (0, 0)))
    taps = []
    for dy in range(3):
        for dx in range(3):
            sl = xp[dy:dy + HH, dx:dx + WW, :].reshape(HW, IN_C)
            taps.append(sl)
    taps = jnp.stack(taps, axis=0)
    taps = jnp.pad(taps, ((0, 0), (0, HW_PAD - HW), (0, 0)))

    wst = jnp.stack([W1[:, :, dy, dx].T for dy in range(3) for dx in range(3)],
                    axis=0)

    w2 = jnp.concatenate([Wl[:, :, 0, 0].T, Ws[:, :, 0, 0].T], axis=1)
    w2 = jnp.pad(w2, ((0, 0), (0, OUT_COLS - w2.shape[1])))
    b2 = jnp.concatenate([bl, bs])
    b2 = jnp.pad(b2, (0, OUT_COLS - b2.shape[0])).reshape(1, OUT_COLS)
    b1r = b1.reshape(1, MID_C)

    y2 = pl.pallas_call(
        _conv_head_body,
        grid=(9,),
        in_specs=[
            pl.BlockSpec((1, HW_PAD, IN_C), lambda t: (t, 0, 0)),
            pl.BlockSpec((1, IN_C, MID_C), lambda t: (t, 0, 0)),
            pl.BlockSpec((MID_C, OUT_COLS), lambda t: (0, 0)),
            pl.BlockSpec((1, MID_C), lambda t: (0, 0)),
            pl.BlockSpec((1, OUT_COLS), lambda t: (0, 0)),
        ],
        out_specs=pl.BlockSpec((HW_PAD, OUT_COLS), lambda t: (0, 0)),
        out_shape=jax.ShapeDtypeStruct((HW_PAD, OUT_COLS), jnp.float32),
        scratch_shapes=[pltpu.VMEM((HW_PAD, MID_C), jnp.float32)],
        compiler_params=pltpu.CompilerParams(
            dimension_semantics=("arbitrary",)),
    )(taps, wst, w2, b1r, b2)
    return y2


N_FLAT = 22528
N_ROWS = 176
N_BLK = 22
N_SORT = 2048
INT_MIN32 = -2147483648
UK_NEGINF = -2139095041
KEY_MASK = 0x7FFFFFFF
BIG = 1 << 30


def _fkey(x):
    b = jax.lax.bitcast_convert_type(x, jnp.int32)
    return jnp.where(b < 0, b ^ KEY_MASK, b)


def _pick_lane(row, l0):
    li = jax.lax.broadcasted_iota(jnp.int32, (1, 128), 1)
    return jnp.sum(jnp.where(li == l0, row, 0.0))


def _proposal_body(dy_ref, dx_ref, dh_ref, dw_ref, bg_ref, fg_ref,
                   ay1_ref, ax1_ref, ay2_ref, ax2_ref, imgsz_ref,
                   out_ref,
                   ry1_ref, rx1_ref, ry2_ref, rx2_ref, su_ref,
                   rby1_ref, rbx1_ref, rby2_ref, rbx2_ref, karr_ref,
                   ky1_ref, kx1_ref, ky2_ref, kx2_ref, kar_ref):
    img_h = imgsz_ref[0].astype(jnp.float32)
    img_w = imgsz_ref[1].astype(jnp.float32)

    a0, a1, a2, a3 = ay1_ref[...], ax1_ref[...], ay2_ref[...], ax2_ref[...]
    src_h = a2 - a0
    src_w = a3 - a1
    src_cy = a0 + 0.5 * src_h
    src_cx = a1 + 0.5 * src_w
    cy = dy_ref[...] * src_h + src_cy
    cx = dx_ref[...] * src_w + src_cx
    h = jnp.exp(dh_ref[...]) * src_h
    w = jnp.exp(dw_ref[...]) * src_w
    y1 = jnp.clip(cy - 0.5 * h, 0.0, img_h)
    x1 = jnp.clip(cx - 0.5 * w, 0.0, img_w)
    y2 = jnp.clip(cy + 0.5 * h, 0.0, img_h)
    x2 = jnp.clip(cx + 0.5 * w, 0.0, img_w)
    ry1_ref[...] = y1
    rx1_ref[...] = x1
    ry2_ref[...] = y2
    rx2_ref[...] = x2
    b, f = bg_ref[...], fg_ref[...]
    m = jnp.maximum(b, f)
    e0 = jnp.exp(b - m)
    e1 = jnp.exp(f - m)
    fgs = e1 / (e0 + e1)
    valid0 = jnp.logical_and(y2 - y1 >= MIN_SIZE, x2 - x1 >= MIN_SIZE)
    sc = jnp.where(valid0, fgs, -jnp.inf)
    row_i = jax.lax.broadcasted_iota(jnp.int32, (N_ROWS, 128), 0)
    lane_i = jax.lax.broadcasted_iota(jnp.int32, (N_ROWS, 128), 1)
    flat = row_i * 128 + lane_i
    real = flat < HW * N_ANCHOR
    su = jnp.where(real, _fkey(sc), INT_MIN32)
    su_ref[...] = su

    flat16 = (jax.lax.broadcasted_iota(jnp.int32, (16, 128), 0) * 128
              + jax.lax.broadcasted_iota(jnp.int32, (16, 128), 1))
    karr_ref[...] = jnp.full((16, 128), INT_MIN32, jnp.int32)
    zeros16 = jnp.zeros((16, 128), jnp.float32)
    ky1_ref[...] = zeros16
    kx1_ref[...] = zeros16
    ky2_ref[...] = zeros16
    kx2_ref[...] = zeros16
    kar_ref[...] = zeros16
    rby1_ref[...] = zeros16
    rbx1_ref[...] = zeros16
    rby2_ref[...] = zeros16
    rbx2_ref[...] = zeros16

    li8 = jax.lax.broadcasted_iota(jnp.int32, (8, 128), 1)
    si8 = jax.lax.broadcasted_iota(jnp.int32, (8, 128), 0)
    li1 = jax.lax.broadcasted_iota(jnp.int32, (1, 128), 1)
    r22 = jax.lax.broadcasted_iota(jnp.int32, (N_BLK, 128), 0)

    su3 = su.reshape(N_BLK, 8, 128)
    fl3 = flat.reshape(N_BLK, 8, 128)
    bm0 = jnp.max(su3, axis=1)
    cf0 = jnp.min(jnp.where(su3 == bm0[:, None, :], fl3, BIG), axis=1)

    def body(r, carry):
        nk_b, bm, cf = carry
        g = jnp.max(bm)
        f0 = jnp.min(jnp.where(bm == g, cf, BIG))
        rr = f0 // 128
        l0 = f0 - rr * 128
        b0 = rr // 8
        b8 = b0 * 8
        lmask = li1 == l0
        su_ref[pl.ds(rr, 1), :] = jnp.where(lmask, INT_MIN32,
                                            su_ref[pl.ds(rr, 1), :])
        blk = su_ref[pl.ds(b8, 8), :]
        bflat = (b8 + si8) * 128 + li8
        nv = jnp.max(blk, axis=0, keepdims=True)
        nf = jnp.min(jnp.where(blk == nv, bflat, BIG), axis=0, keepdims=True)
        rowmask = r22 == b0
        bm = jnp.where(rowmask, jnp.broadcast_to(nv, (N_BLK, 128)), bm)
        cf = jnp.where(rowmask, jnp.broadcast_to(nf, (N_BLK, 128)), cf)
        by1 = _pick_lane(ry1_ref[pl.ds(rr, 1), :], l0)
        bx1 = _pick_lane(rx1_ref[pl.ds(rr, 1), :], l0)
        by2 = _pick_lane(ry2_ref[pl.ds(rr, 1), :], l0)
        bx2 = _pick_lane(rx2_ref[pl.ds(rr, 1), :], l0)
        barea = (by2 - by1) * (bx2 - bx1)
        tly = jnp.maximum(ky1_ref[...], by1)
        tlx = jnp.maximum(kx1_ref[...], bx1)
        bry = jnp.minimum(ky2_ref[...], by2)
        brx = jnp.minimum(kx2_ref[...], bx2)
        why = jnp.maximum(bry - tly, 0.0)
        whx = jnp.maximum(brx - tlx, 0.0)
        inter = why * whx
        iou = inter / (kar_ref[...] + barea - inter + 1e-9)
        viol = jnp.max(jnp.where(iou > NMS_T, 1.0, 0.0))
        keep = viol == 0.0
        rmask = flat16 == r
        karr_ref[...] = jnp.where(rmask, jnp.where(keep, g, UK_NEGINF),
                                  karr_ref[...])
        rby1_ref[...] = jnp.where(rmask, by1, rby1_ref[...])
        rbx1_ref[...] = jnp.where(rmask, bx1, rbx1_ref[...])
        rby2_ref[...] = jnp.where(rmask, by2, rby2_ref[...])
        rbx2_ref[...] = jnp.where(rmask, bx2, rbx2_ref[...])
        amask = jnp.logical_and(flat16 == nk_b, keep)
        ky1_ref[...] = jnp.where(amask, by1, ky1_ref[...])
        kx1_ref[...] = jnp.where(amask, bx1, kx1_ref[...])
        ky2_ref[...] = jnp.where(amask, by2, ky2_ref[...])
        kx2_ref[...] = jnp.where(amask, bx2, kx2_ref[...])
        kar_ref[...] = jnp.where(amask, barea, kar_ref[...])
        nk_b = nk_b + jnp.where(keep, 1, 0)
        return nk_b, bm, cf

    def body8x(i, carry):
        for u in range(8):
            carry = body(8 * i + u, carry)
        return carry

    jax.lax.fori_loop(0, N_PRE // 8, body8x,
                      (jnp.zeros((1, 128), jnp.int32), bm0, cf0))

    def body2(t, dummy):
        k = karr_ref[...]
        g2 = jnp.max(k)
        fr = jnp.min(jnp.where(k == g2, flat16, BIG))
        rr2 = fr // 128
        ll2 = fr - rr2 * 128
        by1 = _pick_lane(rby1_ref[pl.ds(rr2, 1), :], ll2)
        bx1 = _pick_lane(rbx1_ref[pl.ds(rr2, 1), :], ll2)
        by2 = _pick_lane(rby2_ref[pl.ds(rr2, 1), :], ll2)
        bx2 = _pick_lane(rbx2_ref[pl.ds(rr2, 1), :], ll2)
        orow = jnp.where(li1 == 0, by1,
                         jnp.where(li1 == 1, bx1,
                                   jnp.where(li1 == 2, by2,
                                             jnp.where(li1 == 3, bx2, 0.0))))
        out_ref[pl.ds(t, 1), :] = orow
        karr_ref[...] = jnp.where(flat16 == fr, INT_MIN32, karr_ref[...])
        return dummy

    def body2x2(i, dummy):
        body2(2 * i, dummy)
        return body2(2 * i + 1, dummy)

    jax.lax.fori_loop(0, N_POST // 2, body2x2, jnp.int32(0))


def _pad_rows(v):
    return jnp.pad(v.reshape(-1), (0, N_FLAT - HW * N_ANCHOR)).reshape(N_ROWS, 128)


_ANCH_PLANES = tuple(
    np.pad(_ANCHORS[:, c], (0, N_FLAT - HW * N_ANCHOR)).reshape(N_ROWS, 128)
    for c in range(4)
)


def _proposal_call(y2, img_size):
    locs2d = y2[:HW, 0:36]
    dy = _pad_rows(locs2d[:, 0::4])
    dx = _pad_rows(locs2d[:, 1::4])
    dh = _pad_rows(locs2d[:, 2::4])
    dw = _pad_rows(locs2d[:, 3::4])
    sc2d = y2[:HW, 36:54]
    bg = _pad_rows(sc2d[:, 0::2])
    fg = _pad_rows(sc2d[:, 1::2])
    anch = [jnp.asarray(p) for p in _ANCH_PLANES]

    big = pltpu.VMEM((N_ROWS, 128), jnp.float32)
    small = pltpu.VMEM((16, 128), jnp.float32)
    smalli = pltpu.VMEM((16, 128), jnp.int32)
    out = pl.pallas_call(
        _proposal_body,
        in_specs=[pl.BlockSpec((N_ROWS, 128), lambda: (0, 0))] * 10
        + [pl.BlockSpec(memory_space=pltpu.SMEM)],
        out_specs=pl.BlockSpec((304, 128), lambda: (0, 0)),
        out_shape=jax.ShapeDtypeStruct((304, 128), jnp.float32),
        scratch_shapes=[big, big, big, big,
                        pltpu.VMEM((N_ROWS, 128), jnp.int32),
                        small, small, small, small, smalli,
                        small, small, small, small, small],
    )(dy, dx, dh, dw, bg, fg, *anch, img_size)
    return out[:N_POST, 0:4]


def _pad_rows(v):
    return jnp.pad(v.reshape(-1), (0, N_FLAT - HW * N_ANCHOR)).reshape(N_ROWS, 128)


_ANCH_PLANES = tuple(
    np.pad(_ANCHORS[:, c], (0, N_FLAT - HW * N_ANCHOR)).reshape(N_ROWS, 128)
    for c in range(4)
)


def kernel(x, W1, b1, Ws, bs, Wl, bl, img_size):
    n = x.shape[0]
    y2 = _conv_head(x, W1, b1, Ws, bs, Wl, bl)
    locs = y2[:HW, 0:36]
    scores_raw = y2[:HW, 36:54]

    rpn_locs = locs.reshape(n, HH * WW * N_ANCHOR, 4)
    rpn_scores = scores_raw.reshape(n, HH * WW * N_ANCHOR, 2)

    rois = _proposal_call(y2, img_size)
    anchor = jnp.asarray(_ANCHORS)
    roi_indices = jnp.zeros((N_POST,), dtype=jnp.int32)
    return rpn_locs, rpn_scores, rois, roi_indices, anchor
